# baseline (speedup 1.0000x reference)
import functools
import math

import jax
import jax.numpy as jnp
from jax import lax
from jax.experimental import pallas as pl
from jax.experimental.pallas import tpu as pltpu
from jax.experimental.pallas import tpu_sc as plsc

B = 2
L = 4096
DM = 1024
NH = 16
DK = 64
DFF = 4096
NR = 2
BL = 64
NB = L // BL
G = NR * B * NH
CB = 4
RPC = CB * BL
UN = RPC + BL
VW = 128
LT = 512
TD = 512
NCHK = L // 128


def _bdot(a, b):
    return jnp.dot(a.astype(jnp.bfloat16), b.astype(jnp.bfloat16),
                   preferred_element_type=jnp.float32)


def _proj_body(x_ref, wq_ref, bq_ref, wv_ref, bv_ref, qv_ref):
    x = x_ref[0].astype(jnp.bfloat16)
    q = jnp.dot(x, wq_ref[...], preferred_element_type=jnp.float32) + bq_ref[...]
    v = jnp.dot(x, wv_ref[...], preferred_element_type=jnp.float32) + bv_ref[...]
    qv_ref[0] = jnp.concatenate(
        [q.reshape(LT, NH, DK), v.reshape(LT, NH, DK)], axis=-1)


def _proj_call(x2, Wq, bq, Wv, bv):
    return pl.pallas_call(
        _proj_body,
        grid=(B, L // LT),
        in_specs=[
            pl.BlockSpec((1, LT, DM), lambda b, t: (b, t, 0)),
            pl.BlockSpec((DM, DM), lambda b, t: (0, 0)),
            pl.BlockSpec((1, DM), lambda b, t: (0, 0)),
            pl.BlockSpec((DM, DM), lambda b, t: (0, 0)),
            pl.BlockSpec((1, DM), lambda b, t: (0, 0)),
        ],
        out_specs=pl.BlockSpec((1, LT, NH, VW), lambda b, t: (b, t, 0, 0)),
        out_shape=jax.ShapeDtypeStruct((B, L, NH, VW), jnp.float32),
    )(x2, Wq.astype(jnp.bfloat16), bq.reshape(1, DM),
      Wv.astype(jnp.bfloat16), bv.reshape(1, DM))


def _sort_body(h_ref, u_ref):
    h = h_ref[0]
    iota_nb = lax.broadcasted_iota(jnp.int32, (128, NB), 1)
    r_iota = lax.broadcasted_iota(jnp.int32, (128, 128), 0)
    c_iota = lax.broadcasted_iota(jnp.int32, (128, 128), 1)
    tril = (c_iota < r_iota).astype(jnp.float32)

    total = jnp.zeros((1, NB), jnp.float32)
    for c in range(NCHK):
        oh = (h[:, c:c + 1] == iota_nb).astype(jnp.float32)
        total = total + jnp.sum(oh, axis=0, keepdims=True)
    inc = total
    for k in (1, 2, 4, 8, 16, 32):
        inc = inc + jnp.concatenate(
            [jnp.zeros((1, k), jnp.float32), inc[:, :NB - k]], axis=1)
    starts = inc - total

    run = jnp.zeros((1, NB), jnp.float32)
    cols = []
    for c in range(NCHK):
        oh = (h[:, c:c + 1] == iota_nb).astype(jnp.float32)
        ex = jnp.dot(tril, oh, preferred_element_type=jnp.float32) + run
        cols.append(jnp.sum(oh * (starts + ex), axis=1, keepdims=True))
        run = run + jnp.sum(oh, axis=0, keepdims=True)
    u_ref[0] = jnp.concatenate(cols, axis=1).astype(jnp.int32)


def _sort_call(hash_t):
    return pl.pallas_call(
        _sort_body,
        grid=(G,),
        in_specs=[pl.BlockSpec((1, 128, NCHK), lambda g: (g, 0, 0))],
        out_specs=pl.BlockSpec((1, 128, NCHK), lambda g: (g, 0, 0)),
        out_shape=jax.ShapeDtypeStruct((G, 128, NCHK), jnp.int32),
    )(hash_t)


def _task_ids(wid, t):
    g = wid * 2 + t
    b = (g // NH) % B
    h = g % NH
    return g, b, h


NBUF = 4


def _scp_body(undo_hbm, qv_hbm, qvs_hbm, idx_hbm,
              undo_v, idx_v, gidx_v, b0, b1, b2, b3,
              g0, g1, g2, g3, w0, w1, w2, w3):
    bufs = (b0, b1, b2, b3)
    gsems = (g0, g1, g2, g3)
    wsems = (w0, w1, w2, w3)
    wid = lax.axis_index("s") * 2 + lax.axis_index("c")
    for t in range(2):
        g, b, h = _task_ids(wid, t)
        base = b * (L * NH) + h
        pltpu.sync_copy(undo_hbm.at[g], undo_v)

        def inv_body(i, _):
            u = undo_v[pl.ds(i * 16, 16)]
            vals = lax.iota(jnp.int32, 16) + i * 16
            plsc.store_scatter(idx_v, [u], vals)
            return 0
        lax.fori_loop(0, L // 16, inv_body, 0)

        pltpu.sync_copy(idx_v, idx_hbm.at[g])

        def gx_body(i, _):
            ix = idx_v[pl.ds(i * 16, 16)]
            gidx_v[i // 8, pl.ds((i % 8) * 16, 16)] = ix * NH + base
            return 0
        lax.fori_loop(0, L // 16, gx_body, 0)

        def gat_body(i, _):
            cps = []
            for k in range(NBUF):
                c = i * NBUF + k
                cps.append(pltpu.async_copy(
                    qv_hbm.at[gidx_v.at[c]], bufs[k], gsems[k]))
            wps = []
            for k in range(NBUF):
                c = i * NBUF + k
                cps[k].wait()
                wps.append(pltpu.async_copy(
                    bufs[k], qvs_hbm.at[g, pl.ds(c * 128, 128)], wsems[k]))
            for k in range(NBUF):
                wps[k].wait()
            return 0
        lax.fori_loop(0, NCHK // NBUF, gat_body, 0)


def _scp_call(undo, qv_flat):
    mesh = plsc.VectorSubcoreMesh(core_axis_name="c", subcore_axis_name="s")
    f = functools.partial(
        pl.kernel,
        out_type=[
            jax.ShapeDtypeStruct((G, L, VW), jnp.float32),
            jax.ShapeDtypeStruct((G, L), jnp.int32),
        ],
        mesh=mesh,
        scratch_types=(
            [pltpu.VMEM((L,), jnp.int32),
             pltpu.VMEM((L,), jnp.int32),
             pltpu.VMEM((NCHK, 128), jnp.int32)]
            + [pltpu.VMEM((128, VW), jnp.float32)] * NBUF
            + [pltpu.SemaphoreType.DMA] * (2 * NBUF)
        ),
        compiler_params=pltpu.CompilerParams(needs_layout_passes=False),
    )(_scp_body)
    return f(undo, qv_flat)


def _attn_body(qv_ref, ps_ref, pl_ref, o_ref):
    qv = qv_ref[0]
    scale = 1.0 / math.sqrt(DK)
    for j in range(L // RPC):
        lo = j * RPC
        qc = qv[lo:lo + RPC, :DK]
        if j == 0:
            ku = jnp.concatenate([qv[L - BL:], qv[:RPC]], axis=0)
            kp = jnp.concatenate(
                [pl_ref[0, :, L - BL:], pl_ref[0, :, :RPC]], axis=1)
        else:
            ku = qv[lo - BL:lo + RPC]
            kp = pl_ref[0, :, lo - BL:lo + RPC]
        kq = ku[:, :DK]
        vu = ku[:, DK:]
        nrm = jnp.sqrt(jnp.sum(kq * kq, axis=-1, keepdims=True)) + 1e-9
        kn = kq / nrm
        qp = ps_ref[0, lo:lo + RPC, :]
        d = kp - qp
        rb = lax.broadcasted_iota(jnp.int32, (RPC, UN), 0) // BL
        cb = lax.broadcasted_iota(jnp.int32, (RPC, UN), 1) // BL
        valid = (cb == rb) | (cb == rb + 1)
        s = lax.dot_general(
            qc.astype(jnp.bfloat16), kn.astype(jnp.bfloat16),
            (((1,), (1,)), ((), ())),
            preferred_element_type=jnp.float32) * scale
        s = jnp.where(d > 0, -1e9, s)
        s = jnp.where(d == 0, -1e5, s)
        s = jnp.where(valid, s, -jnp.inf)
        m = jnp.max(s, axis=-1, keepdims=True)
        e = jnp.exp(s - m)
        se = jnp.sum(e, axis=-1, keepdims=True)
        p = e * (1.0 / se)
        lse = m + jnp.log(se)
        o = _bdot(p, vu)
        pad = jnp.zeros((RPC, VW - DK - 1), jnp.float32)
        o_ref[0, lo:lo + RPC, :] = jnp.concatenate([o, lse, pad], axis=-1)


def _attn_call(qvs, idx_s, idx_l):
    return pl.pallas_call(
        _attn_body,
        grid=(G,),
        in_specs=[
            pl.BlockSpec((1, L, VW), lambda g: (g, 0, 0)),
            pl.BlockSpec((1, L, 1), lambda g: (g, 0, 0)),
            pl.BlockSpec((1, 1, L), lambda g: (g, 0, 0)),
        ],
        out_specs=pl.BlockSpec((1, L, VW), lambda g: (g, 0, 0)),
        out_shape=jax.ShapeDtypeStruct((G, L, VW), jnp.float32),
    )(qvs, idx_s, idx_l)


def _scs_body(oe_hbm, idx_hbm, onat_hbm, idx_v, gidx_v,
              b0, b1, b2, b3, g0, g1, g2, g3, w0, w1, w2, w3):
    bufs = (b0, b1, b2, b3)
    gsems = (g0, g1, g2, g3)
    wsems = (w0, w1, w2, w3)
    wid = lax.axis_index("s") * 2 + lax.axis_index("c")
    for t in range(2):
        g, b, h = _task_ids(wid, t)
        r = g // (B * NH)
        base = (r * B + b) * (L * NH) + h
        pltpu.sync_copy(idx_hbm.at[g], idx_v)

        def gx_body(i, _):
            ix = idx_v[pl.ds(i * 16, 16)]
            gidx_v[i // 8, pl.ds((i % 8) * 16, 16)] = ix * NH + base
            return 0
        lax.fori_loop(0, L // 16, gx_body, 0)

        def sc_body(i, _):
            cps = []
            for k in range(NBUF):
                c = i * NBUF + k
                cps.append(pltpu.async_copy(
                    oe_hbm.at[g, pl.ds(c * 128, 128)], bufs[k], gsems[k]))
            wps = []
            for k in range(NBUF):
                c = i * NBUF + k
                cps[k].wait()
                wps.append(pltpu.async_copy(
                    bufs[k], onat_hbm.at[gidx_v.at[c]], wsems[k]))
            for k in range(NBUF):
                wps[k].wait()
            return 0
        lax.fori_loop(0, NCHK // NBUF, sc_body, 0)


def _scs_call(oext, idx):
    mesh = plsc.VectorSubcoreMesh(core_axis_name="c", subcore_axis_name="s")
    f = functools.partial(
        pl.kernel,
        out_type=[jax.ShapeDtypeStruct((NR * B * L * NH, VW), jnp.float32)],
        mesh=mesh,
        scratch_types=(
            [pltpu.VMEM((L,), jnp.int32),
             pltpu.VMEM((NCHK, 128), jnp.int32)]
            + [pltpu.VMEM((128, VW), jnp.float32)] * NBUF
            + [pltpu.SemaphoreType.DMA] * (2 * NBUF)
        ),
        compiler_params=pltpu.CompilerParams(needs_layout_passes=False),
    )(_scs_body)
    return f(oext, idx)[0]


def _ln(x, g, b):
    mu = jnp.mean(x, axis=-1, keepdims=True)
    var = jnp.mean((x - mu) ** 2, axis=-1, keepdims=True)
    return (x - mu) / jnp.sqrt(var + 1e-5) * g + b


def _ffn_body(x1_ref, x2_ref, w1_ref, b1_ref, w2_ref, b2_ref, lg_ref, y2_ref):
    x1 = x1_ref[0]
    hmid = jnp.maximum(
        jnp.dot(x1.astype(jnp.bfloat16), w1_ref[...],
                preferred_element_type=jnp.float32) + b1_ref[...], 0.0)
    f = jnp.dot(hmid.astype(jnp.bfloat16), w2_ref[...],
                preferred_element_type=jnp.float32) + b2_ref[...]
    y2_ref[0] = x2_ref[0] + _ln(f, lg_ref[0:1], lg_ref[1:2])


def _ffn_call(x1, x2, w1, b1, w2, b2, lg):
    return pl.pallas_call(
        _ffn_body,
        grid=(B, L // TD),
        in_specs=[
            pl.BlockSpec((1, TD, DM), lambda b, t: (b, t, 0)),
            pl.BlockSpec((1, TD, DM), lambda b, t: (b, t, 0)),
            pl.BlockSpec((DM, DFF), lambda b, t: (0, 0)),
            pl.BlockSpec((1, DFF), lambda b, t: (0, 0)),
            pl.BlockSpec((DFF, DM), lambda b, t: (0, 0)),
            pl.BlockSpec((1, DM), lambda b, t: (0, 0)),
            pl.BlockSpec((2, DM), lambda b, t: (0, 0)),
        ],
        out_specs=pl.BlockSpec((1, TD, DM), lambda b, t: (b, t, 0)),
        out_shape=jax.ShapeDtypeStruct((B, L, DM), jnp.float32),
    )(x1, x2, w1.astype(jnp.bfloat16), b1.reshape(1, DFF),
      w2.astype(jnp.bfloat16), b2.reshape(1, DM), lg)


def _comb_body(on_ref, x1_ref, wo_ref, bo_ref, lf_ref, y1_ref):
    ob0 = on_ref[0, 0]
    ob1 = on_ref[1, 0]
    l0 = ob0[:, :, DK:DK + 1]
    l1 = ob1[:, :, DK:DK + 1]
    m = jnp.maximum(l0, l1)
    e0 = jnp.exp(l0 - m)
    e1 = jnp.exp(l1 - m)
    attn = (e0 * ob0[:, :, :DK] + e1 * ob1[:, :, :DK]) / (e0 + e1)
    a = jnp.dot(attn.reshape(TD, DM).astype(jnp.bfloat16), wo_ref[...],
                preferred_element_type=jnp.float32) + bo_ref[...]
    y1_ref[0] = x1_ref[0] + _ln(a, lf_ref[0:1], lf_ref[1:2])


def _comb_call(onat, x1, Wo, bo, lf):
    return pl.pallas_call(
        _comb_body,
        grid=(B, L // TD),
        in_specs=[
            pl.BlockSpec((NR, 1, TD, NH, VW), lambda b, t: (0, b, t, 0, 0)),
            pl.BlockSpec((1, TD, DM), lambda b, t: (b, t, 0)),
            pl.BlockSpec((DM, DM), lambda b, t: (0, 0)),
            pl.BlockSpec((1, DM), lambda b, t: (0, 0)),
            pl.BlockSpec((2, DM), lambda b, t: (0, 0)),
        ],
        out_specs=pl.BlockSpec((1, TD, DM), lambda b, t: (b, t, 0)),
        out_shape=jax.ShapeDtypeStruct((B, L, DM), jnp.float32),
    )(onat, x1, Wo.astype(jnp.bfloat16), bo.reshape(1, DM), lf)


def kernel(x1, x2, Wq, bq, Wv, bv, Wo, bo, w1, b1, w2, b2,
           ln_f_g, ln_f_b, ln_g_g, ln_g_b):
    key = jax.random.key(42)
    q = (x2 @ Wq + bq).reshape(B, L, NH, DK).transpose(0, 2, 1, 3)
    qn = q / (jnp.linalg.norm(q, axis=-1, keepdims=True) + 1e-9)
    hs = []
    for r in range(NR):
        rk = jax.random.fold_in(key, r)
        R = jax.random.normal(rk, (DK, NB // 2), dtype=jnp.float32)
        R = R / (jnp.linalg.norm(R, axis=0, keepdims=True) + 1e-9)
        proj = jnp.einsum('bhld,dk->bhlk', qn, R)
        hs.append(jnp.argmax(jnp.concatenate([proj, -proj], axis=-1), axis=-1))
    hash_t = (jnp.stack(hs, 0).astype(jnp.int32)
              .reshape(G, NCHK, 128).transpose(0, 2, 1))

    qv = _proj_call(x2, Wq, bq, Wv, bv)
    undo_t = _sort_call(hash_t)
    undo = undo_t.transpose(0, 2, 1).reshape(G, L)
    qvs, idx = _scp_call(undo, qv.reshape(B * L * NH, VW))
    y2 = _ffn_call(x1, x2, w1, b1, w2, b2, jnp.stack([ln_g_g, ln_g_b]))
    oext = _attn_call(qvs, idx.reshape(G, L, 1), idx.reshape(G, 1, L))
    onat = _scs_call(oext, idx)
    y1 = _comb_call(onat.reshape(NR, B, L, NH, VW), x1, Wo, bo,
                    jnp.stack([ln_f_g, ln_f_b]))
    return (y1, y2)


# speedup vs baseline: 1.0197x; 1.0197x over previous
import functools
import math

import jax
import jax.numpy as jnp
from jax import lax
from jax.experimental import pallas as pl
from jax.experimental.pallas import tpu as pltpu
from jax.experimental.pallas import tpu_sc as plsc

B = 2
L = 4096
DM = 1024
NH = 16
DK = 64
DFF = 4096
NR = 2
BL = 64
NB = L // BL
G = NR * B * NH
CB = 4
RPC = CB * BL
UN = RPC + BL
VW = 128
LT = 512
TD = 256
NCHK = L // 128


def _bdot(a, b):
    return jnp.dot(a.astype(jnp.bfloat16), b.astype(jnp.bfloat16),
                   preferred_element_type=jnp.float32)


def _proj_body(x_ref, wq_ref, bq_ref, wv_ref, bv_ref, qv_ref):
    x = x_ref[0]
    q = _bdot(x, wq_ref[...]) + bq_ref[...]
    v = _bdot(x, wv_ref[...]) + bv_ref[...]
    qv_ref[0] = jnp.concatenate(
        [q.reshape(LT, NH, DK), v.reshape(LT, NH, DK)], axis=-1)


def _proj_call(x2, Wq, bq, Wv, bv):
    return pl.pallas_call(
        _proj_body,
        grid=(B, L // LT),
        in_specs=[
            pl.BlockSpec((1, LT, DM), lambda b, t: (b, t, 0)),
            pl.BlockSpec((DM, DM), lambda b, t: (0, 0)),
            pl.BlockSpec((1, DM), lambda b, t: (0, 0)),
            pl.BlockSpec((DM, DM), lambda b, t: (0, 0)),
            pl.BlockSpec((1, DM), lambda b, t: (0, 0)),
        ],
        out_specs=pl.BlockSpec((1, LT, NH, VW), lambda b, t: (b, t, 0, 0)),
        out_shape=jax.ShapeDtypeStruct((B, L, NH, VW), jnp.float32),
    )(x2, Wq, bq.reshape(1, DM), Wv, bv.reshape(1, DM))


def _sort_body(h_ref, u_ref):
    h = h_ref[0]
    iota_nb = lax.broadcasted_iota(jnp.int32, (128, NB), 1)
    r_iota = lax.broadcasted_iota(jnp.int32, (128, 128), 0)
    c_iota = lax.broadcasted_iota(jnp.int32, (128, 128), 1)
    tril = (c_iota < r_iota).astype(jnp.float32)

    total = jnp.zeros((1, NB), jnp.float32)
    for c in range(NCHK):
        oh = (h[:, c:c + 1] == iota_nb).astype(jnp.float32)
        total = total + jnp.sum(oh, axis=0, keepdims=True)
    inc = total
    for k in (1, 2, 4, 8, 16, 32):
        inc = inc + jnp.concatenate(
            [jnp.zeros((1, k), jnp.float32), inc[:, :NB - k]], axis=1)
    starts = inc - total

    run = jnp.zeros((1, NB), jnp.float32)
    cols = []
    for c in range(NCHK):
        oh = (h[:, c:c + 1] == iota_nb).astype(jnp.float32)
        ex = jnp.dot(tril, oh, preferred_element_type=jnp.float32) + run
        cols.append(jnp.sum(oh * (starts + ex), axis=1, keepdims=True))
        run = run + jnp.sum(oh, axis=0, keepdims=True)
    u_ref[0] = jnp.concatenate(cols, axis=1).astype(jnp.int32)


def _sort_call(hash_t):
    return pl.pallas_call(
        _sort_body,
        grid=(G,),
        in_specs=[pl.BlockSpec((1, 128, NCHK), lambda g: (g, 0, 0))],
        out_specs=pl.BlockSpec((1, 128, NCHK), lambda g: (g, 0, 0)),
        out_shape=jax.ShapeDtypeStruct((G, 128, NCHK), jnp.int32),
    )(hash_t)


def _task_ids(wid, t):
    g = wid * 2 + t
    b = (g // NH) % B
    h = g % NH
    return g, b, h


NBUF = 4


def _scp_body(undo_hbm, qv_hbm, qvs_hbm, idx_hbm,
              undo_v, idx_v, gidx_v, b0, b1, b2, b3,
              g0, g1, g2, g3, w0, w1, w2, w3):
    bufs = (b0, b1, b2, b3)
    gsems = (g0, g1, g2, g3)
    wsems = (w0, w1, w2, w3)
    wid = lax.axis_index("s") * 2 + lax.axis_index("c")
    for t in range(2):
        g, b, h = _task_ids(wid, t)
        base = b * (L * NH) + h
        pltpu.sync_copy(undo_hbm.at[g], undo_v)

        def inv_body(i, _):
            u = undo_v[pl.ds(i * 16, 16)]
            vals = lax.iota(jnp.int32, 16) + i * 16
            plsc.store_scatter(idx_v, [u], vals)
            return 0
        lax.fori_loop(0, L // 16, inv_body, 0)

        pltpu.sync_copy(idx_v, idx_hbm.at[g])

        def gx_body(i, _):
            ix = idx_v[pl.ds(i * 16, 16)]
            gidx_v[i // 8, pl.ds((i % 8) * 16, 16)] = ix * NH + base
            return 0
        lax.fori_loop(0, L // 16, gx_body, 0)

        def gat_body(i, _):
            cps = []
            for k in range(NBUF):
                c = i * NBUF + k
                cps.append(pltpu.async_copy(
                    qv_hbm.at[gidx_v.at[c]], bufs[k], gsems[k]))
            wps = []
            for k in range(NBUF):
                c = i * NBUF + k
                cps[k].wait()
                wps.append(pltpu.async_copy(
                    bufs[k], qvs_hbm.at[g, pl.ds(c * 128, 128)], wsems[k]))
            for k in range(NBUF):
                wps[k].wait()
            return 0
        lax.fori_loop(0, NCHK // NBUF, gat_body, 0)


def _scp_call(undo, qv_flat):
    mesh = plsc.VectorSubcoreMesh(core_axis_name="c", subcore_axis_name="s")
    f = functools.partial(
        pl.kernel,
        out_type=[
            jax.ShapeDtypeStruct((G, L, VW), jnp.float32),
            jax.ShapeDtypeStruct((G, L), jnp.int32),
        ],
        mesh=mesh,
        scratch_types=(
            [pltpu.VMEM((L,), jnp.int32),
             pltpu.VMEM((L,), jnp.int32),
             pltpu.VMEM((NCHK, 128), jnp.int32)]
            + [pltpu.VMEM((128, VW), jnp.float32)] * NBUF
            + [pltpu.SemaphoreType.DMA] * (2 * NBUF)
        ),
        compiler_params=pltpu.CompilerParams(needs_layout_passes=False),
    )(_scp_body)
    return f(undo, qv_flat)


def _attn_body(qv_ref, ps_ref, pl_ref, o_ref):
    qv = qv_ref[0]
    scale = 1.0 / math.sqrt(DK)
    for j in range(L // RPC):
        lo = j * RPC
        qc = qv[lo:lo + RPC, :DK]
        if j == 0:
            ku = jnp.concatenate([qv[L - BL:], qv[:RPC]], axis=0)
            kp = jnp.concatenate(
                [pl_ref[0, :, L - BL:], pl_ref[0, :, :RPC]], axis=1)
        else:
            ku = qv[lo - BL:lo + RPC]
            kp = pl_ref[0, :, lo - BL:lo + RPC]
        kq = ku[:, :DK]
        vu = ku[:, DK:]
        nrm = jnp.sqrt(jnp.sum(kq * kq, axis=-1, keepdims=True)) + 1e-9
        kn = kq / nrm
        qp = ps_ref[0, lo:lo + RPC, :]
        d = kp - qp
        rb = lax.broadcasted_iota(jnp.int32, (RPC, UN), 0) // BL
        cb = lax.broadcasted_iota(jnp.int32, (RPC, UN), 1) // BL
        valid = (cb == rb) | (cb == rb + 1)
        s = lax.dot_general(
            qc.astype(jnp.bfloat16), kn.astype(jnp.bfloat16),
            (((1,), (1,)), ((), ())),
            preferred_element_type=jnp.float32) * scale
        s = jnp.where(d > 0, -1e9, s)
        s = jnp.where(d == 0, -1e5, s)
        s = jnp.where(valid, s, -3e38)
        m = jnp.max(s, axis=-1, keepdims=True)
        e = jnp.exp(s - m)
        se = jnp.sum(e, axis=-1, keepdims=True)
        p = e * (1.0 / se)
        lse = m + jnp.log(se)
        o = _bdot(p, vu)
        pad = jnp.zeros((RPC, VW - DK - 1), jnp.float32)
        o_ref[0, lo:lo + RPC, :] = jnp.concatenate([o, lse, pad], axis=-1)


def _attn_call(qvs, idx_s, idx_l):
    return pl.pallas_call(
        _attn_body,
        grid=(G,),
        in_specs=[
            pl.BlockSpec((1, L, VW), lambda g: (g, 0, 0)),
            pl.BlockSpec((1, L, 1), lambda g: (g, 0, 0)),
            pl.BlockSpec((1, 1, L), lambda g: (g, 0, 0)),
        ],
        out_specs=pl.BlockSpec((1, L, VW), lambda g: (g, 0, 0)),
        out_shape=jax.ShapeDtypeStruct((G, L, VW), jnp.float32),
    )(qvs, idx_s, idx_l)


def _scs_body(oe_hbm, idx_hbm, onat_hbm, idx_v, gidx_v,
              b0, b1, b2, b3, g0, g1, g2, g3, w0, w1, w2, w3):
    bufs = (b0, b1, b2, b3)
    gsems = (g0, g1, g2, g3)
    wsems = (w0, w1, w2, w3)
    wid = lax.axis_index("s") * 2 + lax.axis_index("c")
    for t in range(2):
        g, b, h = _task_ids(wid, t)
        r = g // (B * NH)
        base = (r * B + b) * (L * NH) + h
        pltpu.sync_copy(idx_hbm.at[g], idx_v)

        def gx_body(i, _):
            ix = idx_v[pl.ds(i * 16, 16)]
            gidx_v[i // 8, pl.ds((i % 8) * 16, 16)] = ix * NH + base
            return 0
        lax.fori_loop(0, L // 16, gx_body, 0)

        def sc_body(i, _):
            cps = []
            for k in range(NBUF):
                c = i * NBUF + k
                cps.append(pltpu.async_copy(
                    oe_hbm.at[g, pl.ds(c * 128, 128)], bufs[k], gsems[k]))
            wps = []
            for k in range(NBUF):
                c = i * NBUF + k
                cps[k].wait()
                wps.append(pltpu.async_copy(
                    bufs[k], onat_hbm.at[gidx_v.at[c]], wsems[k]))
            for k in range(NBUF):
                wps[k].wait()
            return 0
        lax.fori_loop(0, NCHK // NBUF, sc_body, 0)


def _scs_call(oext, idx):
    mesh = plsc.VectorSubcoreMesh(core_axis_name="c", subcore_axis_name="s")
    f = functools.partial(
        pl.kernel,
        out_type=[jax.ShapeDtypeStruct((NR * B * L * NH, VW), jnp.float32)],
        mesh=mesh,
        scratch_types=(
            [pltpu.VMEM((L,), jnp.int32),
             pltpu.VMEM((NCHK, 128), jnp.int32)]
            + [pltpu.VMEM((128, VW), jnp.float32)] * NBUF
            + [pltpu.SemaphoreType.DMA] * (2 * NBUF)
        ),
        compiler_params=pltpu.CompilerParams(needs_layout_passes=False),
    )(_scs_body)
    return f(oext, idx)[0]


def _ln(x, g, b):
    mu = jnp.mean(x, axis=-1, keepdims=True)
    var = jnp.mean((x - mu) ** 2, axis=-1, keepdims=True)
    return (x - mu) / jnp.sqrt(var + 1e-5) * g + b


def _ffn_body(x1_ref, x2_ref, w1_ref, b1_ref, w2_ref, b2_ref, lg_ref, y2_ref):
    x1 = x1_ref[0]
    hmid = jnp.maximum(_bdot(x1, w1_ref[...]) + b1_ref[...], 0.0)
    f = _bdot(hmid, w2_ref[...]) + b2_ref[...]
    y2_ref[0] = x2_ref[0] + _ln(f, lg_ref[0:1], lg_ref[1:2])


def _ffn_call(x1, x2, w1, b1, w2, b2, lg):
    return pl.pallas_call(
        _ffn_body,
        grid=(B, L // TD),
        in_specs=[
            pl.BlockSpec((1, TD, DM), lambda b, t: (b, t, 0)),
            pl.BlockSpec((1, TD, DM), lambda b, t: (b, t, 0)),
            pl.BlockSpec((DM, DFF), lambda b, t: (0, 0)),
            pl.BlockSpec((1, DFF), lambda b, t: (0, 0)),
            pl.BlockSpec((DFF, DM), lambda b, t: (0, 0)),
            pl.BlockSpec((1, DM), lambda b, t: (0, 0)),
            pl.BlockSpec((2, DM), lambda b, t: (0, 0)),
        ],
        out_specs=pl.BlockSpec((1, TD, DM), lambda b, t: (b, t, 0)),
        out_shape=jax.ShapeDtypeStruct((B, L, DM), jnp.float32),
    )(x1, x2, w1, b1.reshape(1, DFF), w2, b2.reshape(1, DM), lg)


def _comb_body(on_ref, x1_ref, wo_ref, bo_ref, lf_ref, y1_ref):
    ob0 = on_ref[0, 0]
    ob1 = on_ref[1, 0]
    l0 = ob0[:, :, DK:DK + 1]
    l1 = ob1[:, :, DK:DK + 1]
    m = jnp.maximum(l0, l1)
    e0 = jnp.exp(l0 - m)
    e1 = jnp.exp(l1 - m)
    attn = (e0 * ob0[:, :, :DK] + e1 * ob1[:, :, :DK]) / (e0 + e1)
    a = _bdot(attn.reshape(TD, DM), wo_ref[...]) + bo_ref[...]
    y1_ref[0] = x1_ref[0] + _ln(a, lf_ref[0:1], lf_ref[1:2])


def _comb_call(onat, x1, Wo, bo, lf):
    return pl.pallas_call(
        _comb_body,
        grid=(B, L // TD),
        in_specs=[
            pl.BlockSpec((NR, 1, TD, NH, VW), lambda b, t: (0, b, t, 0, 0)),
            pl.BlockSpec((1, TD, DM), lambda b, t: (b, t, 0)),
            pl.BlockSpec((DM, DM), lambda b, t: (0, 0)),
            pl.BlockSpec((1, DM), lambda b, t: (0, 0)),
            pl.BlockSpec((2, DM), lambda b, t: (0, 0)),
        ],
        out_specs=pl.BlockSpec((1, TD, DM), lambda b, t: (b, t, 0)),
        out_shape=jax.ShapeDtypeStruct((B, L, DM), jnp.float32),
    )(onat, x1, Wo, bo.reshape(1, DM), lf)


def kernel(x1, x2, Wq, bq, Wv, bv, Wo, bo, w1, b1, w2, b2,
           ln_f_g, ln_f_b, ln_g_g, ln_g_b):
    key = jax.random.key(42)
    q = (x2 @ Wq + bq).reshape(B, L, NH, DK).transpose(0, 2, 1, 3)
    qn = q / (jnp.linalg.norm(q, axis=-1, keepdims=True) + 1e-9)
    hs = []
    for r in range(NR):
        rk = jax.random.fold_in(key, r)
        R = jax.random.normal(rk, (DK, NB // 2), dtype=jnp.float32)
        R = R / (jnp.linalg.norm(R, axis=0, keepdims=True) + 1e-9)
        proj = jnp.einsum('bhld,dk->bhlk', qn, R)
        hs.append(jnp.argmax(jnp.concatenate([proj, -proj], axis=-1), axis=-1))
    hash_t = (jnp.stack(hs, 0).astype(jnp.int32)
              .reshape(G, NCHK, 128).transpose(0, 2, 1))

    qv = _proj_call(x2, Wq, bq, Wv, bv)
    undo_t = _sort_call(hash_t)
    undo = undo_t.transpose(0, 2, 1).reshape(G, L)
    qvs, idx = _scp_call(undo, qv.reshape(B * L * NH, VW))
    y2 = _ffn_call(x1, x2, w1, b1, w2, b2, jnp.stack([ln_g_g, ln_g_b]))
    oext = _attn_call(qvs, idx.reshape(G, L, 1), idx.reshape(G, 1, L))
    onat = _scs_call(oext, idx)
    y1 = _comb_call(onat.reshape(NR, B, L, NH, VW), x1, Wo, bo,
                    jnp.stack([ln_f_g, ln_f_b]))
    return (y1, y2)
